# Initial kernel scaffold; baseline (speedup 1.0000x reference)
import jax
import jax.numpy as jnp
from jax.experimental import pallas as pl


def kernel(x, edge_index, W1, b1, W2, b2):
    raise NotImplementedError("write your pallas kernel here")


# speedup vs baseline: 9.0697x; 9.0697x over previous
import functools

import jax
import jax.numpy as jnp
from jax import lax
from jax.experimental import pallas as pl
from jax.experimental.pallas import tpu as pltpu
from jax.experimental.pallas import tpu_sc as plsc

_NC = 2
_NS = 16
_NW = _NC * _NS
_B = 128


def _sc_scatter(g, src3, dst3, zfeat, npad, d, k):
    rpt = npad // _NS
    mesh = plsc.VectorSubcoreMesh(core_axis_name="c", subcore_axis_name="s")

    @functools.partial(
        pl.kernel,
        out_type=jax.ShapeDtypeStruct((_NC, npad, d), jnp.float32),
        mesh=mesh,
        scratch_types=[
            pltpu.VMEM((k, _B), jnp.int32),
            pltpu.VMEM((k, _B), jnp.int32),
            pltpu.VMEM((_B, d), jnp.float32),
            pltpu.VMEM_SHARED((npad, d), jnp.float32),
            pltpu.SemaphoreType.DMA,
        ],
    )
    def scat_kernel(g_hbm, src_hbm, dst_hbm, z_hbm, out_hbm,
                    idx_s, idx_d, rows, acc, sem):
        cid = lax.axis_index("c")
        sid = lax.axis_index("s")
        wid = sid * _NC + cid
        sl = pl.ds(sid * rpt, rpt)

        @pl.when(cid == 0)
        def _():
            pltpu.sync_copy(g_hbm.at[sl], acc.at[sl])

        @pl.when(cid != 0)
        def _():
            pltpu.sync_copy(z_hbm.at[sl], acc.at[sl])

        pltpu.sync_copy(src_hbm.at[wid], idx_s)
        pltpu.sync_copy(dst_hbm.at[wid], idx_d)
        plsc.subcore_barrier()

        @pl.loop(0, k)
        def _(j):
            pltpu.async_copy(g_hbm.at[idx_s.at[j]], rows, sem).wait()
            pltpu.sync_copy(rows, acc.at[idx_d.at[j]], add=True)

        plsc.subcore_barrier()
        pltpu.sync_copy(acc.at[sl], out_hbm.at[cid, sl])

    return scat_kernel(g, src3, dst3, zfeat)


def _dinv_of(degs_blk):
    deg = degs_blk[0, :, 0] + degs_blk[1, :, 0]
    return lax.rsqrt(deg)


def _tc_g1(xpad, W1, degs, npad, d, bm):
    def body(x_ref, w_ref, degs_ref, g_ref):
        h = jnp.dot(x_ref[...], w_ref[...], preferred_element_type=jnp.float32)
        g_ref[...] = h * _dinv_of(degs_ref)[:, None]

    return pl.pallas_call(
        body,
        grid=(npad // bm,),
        in_specs=[
            pl.BlockSpec((bm, d), lambda i: (i, 0)),
            pl.BlockSpec((d, d), lambda i: (0, 0)),
            pl.BlockSpec((_NC, bm, d), lambda i: (0, i, 0)),
        ],
        out_specs=pl.BlockSpec((bm, d), lambda i: (i, 0)),
        out_shape=jax.ShapeDtypeStruct((npad, d), jnp.float32),
    )(xpad, W1, degs)


def _tc_mid(S1, degs, b1, W2, npad, d, bm):
    def body(s_ref, degs_ref, b_ref, w_ref, pen_ref, g2_ref):
        dinv = _dinv_of(degs_ref)[:, None]
        ssum = s_ref[0] + s_ref[1]
        pen = jnp.maximum(ssum * dinv + b_ref[...], 0.0)
        pen_ref[...] = pen
        h2 = jnp.dot(pen, w_ref[...], preferred_element_type=jnp.float32)
        g2_ref[...] = h2 * dinv

    return pl.pallas_call(
        body,
        grid=(npad // bm,),
        in_specs=[
            pl.BlockSpec((_NC, bm, d), lambda i: (0, i, 0)),
            pl.BlockSpec((_NC, bm, d), lambda i: (0, i, 0)),
            pl.BlockSpec((1, d), lambda i: (0, 0)),
            pl.BlockSpec((d, d), lambda i: (0, 0)),
        ],
        out_specs=[
            pl.BlockSpec((bm, d), lambda i: (i, 0)),
            pl.BlockSpec((bm, d), lambda i: (i, 0)),
        ],
        out_shape=[
            jax.ShapeDtypeStruct((npad, d), jnp.float32),
            jax.ShapeDtypeStruct((npad, d), jnp.float32),
        ],
    )(S1, degs, b1, W2)


def _tc_out(S2, degs, b2, npad, d, bm):
    def body(s_ref, degs_ref, b_ref, out_ref):
        dinv = _dinv_of(degs_ref)[:, None]
        out_ref[...] = (s_ref[0] + s_ref[1]) * dinv + b_ref[...]

    return pl.pallas_call(
        body,
        grid=(npad // bm,),
        in_specs=[
            pl.BlockSpec((_NC, bm, d), lambda i: (0, i, 0)),
            pl.BlockSpec((_NC, bm, d), lambda i: (0, i, 0)),
            pl.BlockSpec((1, d), lambda i: (0, 0)),
        ],
        out_specs=pl.BlockSpec((bm, d), lambda i: (i, 0)),
        out_shape=jax.ShapeDtypeStruct((npad, d), jnp.float32),
    )(S2, degs, b2)


def kernel(x, edge_index, W1, b1, W2, b2):
    n, d = x.shape
    e = edge_index.shape[1]
    bm = 256
    npad = -(-n // bm) * bm
    epw = -(-e // _NW)
    k = -(-epw // _B)
    epad = _NW * k * _B

    pad = jnp.full((epad - e,), n, jnp.int32)
    src3 = jnp.concatenate([edge_index[0], pad]).reshape(_NW, k, _B)
    dst3 = jnp.concatenate([edge_index[1], pad]).reshape(_NW, k, _B)

    xpad = jnp.zeros((npad, d), x.dtype).at[:n].set(x)
    ones_feat = jnp.ones((npad, d), jnp.float32)
    zfeat = jnp.zeros((npad, d), jnp.float32)
    b1r = b1.reshape(1, d)
    b2r = b2.reshape(1, d)

    degs = _sc_scatter(ones_feat, src3, dst3, zfeat, npad, d, k)
    g1 = _tc_g1(xpad, W1, degs, npad, d, bm)
    S1 = _sc_scatter(g1, src3, dst3, zfeat, npad, d, k)
    pen_pad, g2 = _tc_mid(S1, degs, b1r, W2, npad, d, bm)
    S2 = _sc_scatter(g2, src3, dst3, zfeat, npad, d, k)
    out_pad = _tc_out(S2, degs, b2r, npad, d, bm)
    return (out_pad[:n], pen_pad[:n])
